# baseline (speedup 1.0000x reference)
import functools

import jax
import jax.numpy as jnp
import numpy as np
from jax import lax
from jax.experimental import pallas as pl
from jax.experimental.pallas import tpu as pltpu
from jax.experimental.pallas import tpu_sc as plsc

NPTS = 100000
BLKC = 6400
NPAD = 102400
NBLK = NPAD // BLKC
NSORT = 131072
SROWS = NSORT // 128

_f32 = jnp.float32


def _c(v):
    return _f32(np.float32(v))


def _erfc_xla(q):
    one = _c(1.0)
    q2 = q * q
    p = q2 * _c(7.85386146e-05)
    p = p + _c(-0.000801019371)
    p = p * q2
    p = p + _c(0.00518832775)
    p = p * q2
    p = p + _c(-0.0268538129)
    p = p * q2
    p = p + _c(0.112835854)
    p = p * q2
    p = p + _c(-0.37612626)
    p = p * q2
    p = p + _c(1.12837911)
    r_small = one - q * p
    aq = jnp.abs(q)
    nq2 = -q2
    e = jnp.exp(nq2)
    g = e * (one / aq)
    z = one / q2
    pa = z * _c(0.0232682)
    pa = pa + _c(-0.138703942)
    pa = pa * z
    pa = pa + _c(0.368742466)
    pa = pa * z
    pa = pa + _c(-0.582473278)
    pa = pa * z
    pa = pa + _c(0.621000469)
    pa = pa * z
    pa = pa + _c(-0.494451523)
    pa = pa * z
    pa = pa + _c(0.340488)
    pa = pa * z
    pa = pa + _c(-0.274112701)
    pa = pa * z
    pa = pa + _c(0.563825965)
    pb = z * _c(-10.477664)
    pb = pb + _c(12.9772)
    pb = pb * z
    pb = pb + _c(-7.49551868)
    pb = pb * z
    pb = pb + _c(2.92101908)
    pb = pb * z
    pb = pb + _c(-1.01526523)
    pb = pb * z
    pb = pb + _c(0.42184633)
    pb = pb * z
    pb = pb + _c(-0.282076746)
    pb = pb * z
    pb = pb + _c(0.564189494)
    sel_poly = jnp.where(aq < _c(2.0), pa, pb)
    r_large = g * sel_poly
    r_large = jnp.where(nq2 < _c(-88.7228394), _c(0.0), r_large)
    r_neg = jnp.where(q < _c(0.0), _c(2.0) - r_large, r_large)
    return jnp.where(aq < _c(1.0), r_small, r_neg)


def _gelu_xla(x):
    q = (-x) * _c(0.707106769)
    return (x * _c(0.5)) * _erfc_xla(q)


def _lane_tree_sum(acc):
    for sh in (64, 32, 16, 8, 4, 2, 1):
        acc = acc + pltpu.roll(acc, 128 - sh, 1)
    return acc


def _lane_tree_max(acc):
    for sh in (64, 32, 16, 8, 4, 2, 1):
        acc = jnp.maximum(acc, pltpu.roll(acc, 128 - sh, 1))
    return acc


def _colmask(b, shape):
    col = b * BLKC + lax.broadcasted_iota(jnp.int32, shape, 1)
    return col < NPTS


def _k_mm1(xt_ref, w_ref, b_ref, h_ref, m_ref, acc_ref):
    b = pl.program_id(0)
    h = lax.dot_general(
        w_ref[...], xt_ref[...], (((1,), (0,)), ((), ())),
        preferred_element_type=jnp.float32,
    )
    h = h + b_ref[...]
    h_ref[...] = h

    @pl.when(b == 0)
    def _():
        acc_ref[...] = jnp.zeros_like(acc_ref)

    hm = jnp.where(_colmask(b, h.shape), h, _c(0.0))
    for ct in range(BLKC // 128):
        acc_ref[...] += hm[:, 128 * ct:128 * (ct + 1)]

    @pl.when(b == NBLK - 1)
    def _():
        red = _lane_tree_sum(acc_ref[...])
        m_ref[...] = red[:, 0:1] * _c(1e-05)


def _k_var(h_ref, m_ref, d_ref, acc_ref):
    b = pl.program_id(0)

    @pl.when(b == 0)
    def _():
        acc_ref[...] = jnp.zeros_like(acc_ref)

    d = h_ref[...] - m_ref[...]
    sq = d * d
    sq = jnp.where(_colmask(b, sq.shape), sq, _c(0.0))
    for ct in range(BLKC // 128):
        acc_ref[...] += sq[:, 128 * ct:128 * (ct + 1)]

    @pl.when(b == NBLK - 1)
    def _():
        red = _lane_tree_sum(acc_ref[...])
        var = red[:, 0:1] * _c(1e-05)
        d_ref[...] = jnp.sqrt(var + _c(1e-05))


def _k_mm2(h1_ref, m_ref, den_ref, g_ref, be_ref, w_ref, b2_ref,
           h2_ref, m2_ref, acc_ref):
    b = pl.program_id(0)
    x = (h1_ref[...] - m_ref[...]) / den_ref[...] * g_ref[...] + be_ref[...]
    gx = _gelu_xla(x)
    h2 = lax.dot_general(
        w_ref[...], gx, (((1,), (0,)), ((), ())),
        preferred_element_type=jnp.float32,
    )
    h2 = h2 + b2_ref[...]
    h2_ref[...] = h2

    @pl.when(b == 0)
    def _():
        acc_ref[...] = jnp.zeros_like(acc_ref)

    hm = jnp.where(_colmask(b, h2.shape), h2, _c(0.0))
    for ct in range(BLKC // 128):
        acc_ref[...] += hm[:, 128 * ct:128 * (ct + 1)]

    @pl.when(b == NBLK - 1)
    def _():
        red = _lane_tree_sum(acc_ref[...])
        m2_ref[...] = red[:, 0:1] * _c(1e-05)


def _k_scores(h2_ref, m_ref, den_ref, g_ref, be_ref, w3_ref, b3_ref,
              s_ref, mx_ref, acc_ref):
    b = pl.program_id(0)
    x = (h2_ref[...] - m_ref[...]) / den_ref[...] * g_ref[...] + be_ref[...]
    gx = _gelu_xla(x)
    u = lax.dot_general(
        w3_ref[...], gx, (((1,), (0,)), ((), ())),
        preferred_element_type=jnp.float32,
    )
    u = u + b3_ref[...]
    s = _c(1.0) / (jnp.exp(-u) + _c(1.0))
    s_ref[...] = s

    @pl.when(b == 0)
    def _():
        acc_ref[...] = jnp.full_like(acc_ref, -jnp.inf)

    sm = jnp.where(_colmask(b, s.shape), s, -jnp.inf)
    for ct in range(BLKC // 128):
        acc_ref[...] = jnp.maximum(acc_ref[...], sm[:, 128 * ct:128 * (ct + 1)])

    @pl.when(b == NBLK - 1)
    def _():
        red = _lane_tree_max(acc_ref[...])
        mx_ref[...] = red[:, 0:1]


def _k_sort(scores_ref, batch_ref, mx_ref, jt_ref, kt_ref, su_ref, sd_ref,
            ax_ref, ord_ref, keys_ref, idx_ref, pk_ref, pi_ref, iv_ref):
    rows = lax.broadcasted_iota(jnp.int32, (SROWS, 128), 0)
    cols = lax.broadcasted_iota(jnp.int32, (SROWS, 128), 1)
    ivec = rows * 128 + cols
    iv_ref[...] = ivec

    off = mx_ref[0] + _c(10.0)
    nreal_rows = NPAD // 128
    sc = scores_ref[...]
    bt = batch_ref[...].astype(jnp.float32)
    kreal = sc + bt * off
    kreal = jnp.where(ivec[:nreal_rows] < NPTS, kreal, jnp.inf)
    keys_ref[...] = jnp.full((SROWS, 128), jnp.inf, dtype=jnp.float32)
    keys_ref[0:nreal_rows, :] = kreal
    idx_ref[...] = ivec

    def stage(s, _):
        j = jt_ref[s]
        k = kt_ref[s]
        sup = su_ref[s]
        sdn = sd_ref[s]
        is_row = ax_ref[s]
        iv = iv_ref[...]
        maskj = (iv & j) == 0
        asc = (iv & k) == 0

        @pl.when(is_row == 1)
        def _():
            ks = keys_ref[...]
            ix = idx_ref[...]
            pk_ref[...] = jnp.where(
                maskj, pltpu.roll(ks, sup, 0), pltpu.roll(ks, sdn, 0))
            pi_ref[...] = jnp.where(
                maskj, pltpu.roll(ix, sup, 0), pltpu.roll(ix, sdn, 0))

        @pl.when(is_row == 0)
        def _():
            ks = keys_ref[...]
            ix = idx_ref[...]
            pk_ref[...] = jnp.where(
                maskj, pltpu.roll(ks, sup, 1), pltpu.roll(ks, sdn, 1))
            pi_ref[...] = jnp.where(
                maskj, pltpu.roll(ix, sup, 1), pltpu.roll(ix, sdn, 1))

        ks = keys_ref[...]
        ix = idx_ref[...]
        pk = pk_ref[...]
        pi = pi_ref[...]
        partner_less = (pk < ks) | ((pk == ks) & (pi < ix))
        want_min = ~(maskj ^ asc)
        take = ~(partner_less ^ want_min)
        keys_ref[...] = jnp.where(take, pk, ks)
        idx_ref[...] = jnp.where(take, pi, ix)
        return 0

    nstages = 17 * 18 // 2
    lax.fori_loop(0, nstages, stage, 0)
    ord_ref[...] = idx_ref[...]


def _make_inverse_sc():
    info = plsc.get_sparse_core_info()
    nc, ns = info.num_cores, info.num_subcores
    nw = nc * ns
    chunk = NSORT // nw
    mesh = plsc.VectorSubcoreMesh(core_axis_name="c", subcore_axis_name="s")

    @functools.partial(
        pl.kernel,
        mesh=mesh,
        out_type=jax.ShapeDtypeStruct((NSORT,), jnp.int32),
        scratch_types=[
            pltpu.VMEM((chunk,), jnp.int32),
            pltpu.VMEM((chunk,), jnp.int32),
            pltpu.SemaphoreType.DMA,
        ],
    )
    def inv_kernel(order_hbm, ranks_hbm, inv_hbm, idx_v, val_v, sem):
        wid = lax.axis_index("s") * nc + lax.axis_index("c")
        base = wid * chunk
        pltpu.sync_copy(order_hbm.at[pl.ds(base, chunk)], idx_v)
        pltpu.sync_copy(ranks_hbm.at[pl.ds(base, chunk)], val_v)
        pltpu.async_copy(val_v, inv_hbm.at[idx_v], sem).wait()

    return inv_kernel


def _sort_tables():
    js, ks, sups, sdns, axs = [], [], [], [], []
    for klog in range(1, 18):
        for jlog in reversed(range(klog)):
            j = 1 << jlog
            k = 1 << klog
            js.append(j)
            ks.append(k)
            if j >= 128:
                jr = j // 128
                sups.append(SROWS - jr)
                sdns.append(jr)
                axs.append(1)
            else:
                sups.append(128 - j)
                sdns.append(j)
                axs.append(0)
    return (np.array(js, np.int32), np.array(ks, np.int32),
            np.array(sups, np.int32), np.array(sdns, np.int32),
            np.array(axs, np.int32))


def kernel(coord, feat, batch, W1, b1, g1, beta1, W2, b2, g2, beta2, W3, b3):
    xb = jnp.concatenate([coord, feat], axis=1).astype(jnp.bfloat16)
    xt = jnp.pad(xb.T, ((0, 0), (0, NPAD - NPTS)))
    w1t = W1.T
    w2t = W2.T
    w3t = W3.T
    b1c = b1[:, None]
    b2c = b2[:, None]
    g1c = g1[:, None]
    g2c = g2[:, None]
    be1c = beta1[:, None]
    be2c = beta2[:, None]
    b3c = b3[:, None]

    grid = (NBLK,)
    cspec = pl.BlockSpec((64, BLKC), lambda b: (0, b))
    full = pl.BlockSpec((64, 1), lambda b: (0, 0))

    h1t, m1 = pl.pallas_call(
        _k_mm1,
        grid=grid,
        in_specs=[
            pl.BlockSpec((131, BLKC), lambda b: (0, b)),
            pl.BlockSpec((64, 131), lambda b: (0, 0)),
            full,
        ],
        out_specs=[cspec, full],
        out_shape=[
            jax.ShapeDtypeStruct((64, NPAD), jnp.float32),
            jax.ShapeDtypeStruct((64, 1), jnp.float32),
        ],
        scratch_shapes=[pltpu.VMEM((64, 128), jnp.float32)],
    )(xt, w1t, b1c)

    den1 = pl.pallas_call(
        _k_var,
        grid=grid,
        in_specs=[cspec, full],
        out_specs=full,
        out_shape=jax.ShapeDtypeStruct((64, 1), jnp.float32),
        scratch_shapes=[pltpu.VMEM((64, 128), jnp.float32)],
    )(h1t, m1)

    h2t, m2 = pl.pallas_call(
        _k_mm2,
        grid=grid,
        in_specs=[
            cspec, full, full, full, full,
            pl.BlockSpec((64, 64), lambda b: (0, 0)),
            full,
        ],
        out_specs=[cspec, full],
        out_shape=[
            jax.ShapeDtypeStruct((64, NPAD), jnp.float32),
            jax.ShapeDtypeStruct((64, 1), jnp.float32),
        ],
        scratch_shapes=[pltpu.VMEM((64, 128), jnp.float32)],
    )(h1t, m1, den1, g1c, be1c, w2t, b2c)

    den2 = pl.pallas_call(
        _k_var,
        grid=grid,
        in_specs=[cspec, full],
        out_specs=full,
        out_shape=jax.ShapeDtypeStruct((64, 1), jnp.float32),
        scratch_shapes=[pltpu.VMEM((64, 128), jnp.float32)],
    )(h2t, m2)

    rspec = pl.BlockSpec((1, BLKC), lambda b: (0, b))
    rfull = pl.BlockSpec((1, 1), lambda b: (0, 0))
    scorest, mx = pl.pallas_call(
        _k_scores,
        grid=grid,
        in_specs=[
            cspec, full, full, full, full,
            pl.BlockSpec((1, 64), lambda b: (0, 0)),
            rfull,
        ],
        out_specs=[rspec, rfull],
        out_shape=[
            jax.ShapeDtypeStruct((1, NPAD), jnp.float32),
            jax.ShapeDtypeStruct((1, 1), jnp.float32),
        ],
        scratch_shapes=[pltpu.VMEM((1, 128), jnp.float32)],
    )(h2t, m2, den2, g2c, be2c, w3t, b3c)

    jt, kt, su, sd, ax = _sort_tables()
    batch_p = jnp.pad(batch, (0, NPAD - NPTS))
    nsr = NPAD // 128
    order2d = pl.pallas_call(
        _k_sort,
        in_specs=[
            pl.BlockSpec(memory_space=pltpu.VMEM),
            pl.BlockSpec(memory_space=pltpu.VMEM),
            pl.BlockSpec(memory_space=pltpu.SMEM),
            pl.BlockSpec(memory_space=pltpu.SMEM),
            pl.BlockSpec(memory_space=pltpu.SMEM),
            pl.BlockSpec(memory_space=pltpu.SMEM),
            pl.BlockSpec(memory_space=pltpu.SMEM),
            pl.BlockSpec(memory_space=pltpu.SMEM),
        ],
        out_shape=jax.ShapeDtypeStruct((SROWS, 128), jnp.int32),
        scratch_shapes=[
            pltpu.VMEM((SROWS, 128), jnp.float32),
            pltpu.VMEM((SROWS, 128), jnp.int32),
            pltpu.VMEM((SROWS, 128), jnp.float32),
            pltpu.VMEM((SROWS, 128), jnp.int32),
            pltpu.VMEM((SROWS, 128), jnp.int32),
        ],
    )(scorest.reshape(nsr, 128), batch_p.reshape(nsr, 128),
      jnp.asarray(mx).reshape(1), jt, kt, su, sd, ax)

    order_full = order2d.reshape(NSORT)
    ranks = jnp.arange(NSORT, dtype=jnp.int32)
    inv_kernel = _make_inverse_sc()
    inv_full = inv_kernel(order_full, ranks)

    scores = scorest.reshape(NPAD)[:NPTS].reshape(NPTS, 1)
    orders = order_full[:NPTS].reshape(1, NPTS)
    inverses = inv_full[:NPTS].reshape(1, NPTS)
    return (scores, orders, inverses)


# speedup vs baseline: 1.3338x; 1.3338x over previous
import functools

import jax
import jax.numpy as jnp
import numpy as np
from jax import lax
from jax.experimental import pallas as pl
from jax.experimental.pallas import tpu as pltpu
from jax.experimental.pallas import tpu_sc as plsc

NPTS = 100000
BLKC = 6400
NPAD = 102400
NBLK = NPAD // BLKC
NSORT = 131072
SROWS = NSORT // 128

_f32 = jnp.float32


def _c(v):
    return _f32(np.float32(v))


def _erfc_xla(q):
    one = _c(1.0)
    q2 = q * q
    p = q2 * _c(7.85386146e-05)
    p = p + _c(-0.000801019371)
    p = p * q2
    p = p + _c(0.00518832775)
    p = p * q2
    p = p + _c(-0.0268538129)
    p = p * q2
    p = p + _c(0.112835854)
    p = p * q2
    p = p + _c(-0.37612626)
    p = p * q2
    p = p + _c(1.12837911)
    r_small = one - q * p
    aq = jnp.abs(q)
    nq2 = -q2
    e = jnp.exp(nq2)
    g = e * (one / aq)
    z = one / q2
    pa = z * _c(0.0232682)
    pa = pa + _c(-0.138703942)
    pa = pa * z
    pa = pa + _c(0.368742466)
    pa = pa * z
    pa = pa + _c(-0.582473278)
    pa = pa * z
    pa = pa + _c(0.621000469)
    pa = pa * z
    pa = pa + _c(-0.494451523)
    pa = pa * z
    pa = pa + _c(0.340488)
    pa = pa * z
    pa = pa + _c(-0.274112701)
    pa = pa * z
    pa = pa + _c(0.563825965)
    pb = z * _c(-10.477664)
    pb = pb + _c(12.9772)
    pb = pb * z
    pb = pb + _c(-7.49551868)
    pb = pb * z
    pb = pb + _c(2.92101908)
    pb = pb * z
    pb = pb + _c(-1.01526523)
    pb = pb * z
    pb = pb + _c(0.42184633)
    pb = pb * z
    pb = pb + _c(-0.282076746)
    pb = pb * z
    pb = pb + _c(0.564189494)
    sel_poly = jnp.where(aq < _c(2.0), pa, pb)
    r_large = g * sel_poly
    r_large = jnp.where(nq2 < _c(-88.7228394), _c(0.0), r_large)
    r_neg = jnp.where(q < _c(0.0), _c(2.0) - r_large, r_large)
    return jnp.where(aq < _c(1.0), r_small, r_neg)


def _gelu_xla(x):
    q = (-x) * _c(0.707106769)
    return (x * _c(0.5)) * _erfc_xla(q)


def _lane_tree_sum(acc):
    for sh in (64, 32, 16, 8, 4, 2, 1):
        acc = acc + pltpu.roll(acc, 128 - sh, 1)
    return acc


def _lane_tree_max(acc):
    for sh in (64, 32, 16, 8, 4, 2, 1):
        acc = jnp.maximum(acc, pltpu.roll(acc, 128 - sh, 1))
    return acc


def _colmask(b, shape):
    col = b * BLKC + lax.broadcasted_iota(jnp.int32, shape, 1)
    return col < NPTS


def _k_mm1(xt_ref, w_ref, b_ref, h_ref, m_ref, acc_ref):
    b = pl.program_id(0)
    h = lax.dot_general(
        w_ref[...], xt_ref[...], (((1,), (0,)), ((), ())),
        preferred_element_type=jnp.float32,
    )
    h = h + b_ref[...]
    h_ref[...] = h

    @pl.when(b == 0)
    def _():
        acc_ref[...] = jnp.zeros_like(acc_ref)

    hm = jnp.where(_colmask(b, h.shape), h, _c(0.0))
    for ct in range(BLKC // 128):
        acc_ref[...] += hm[:, 128 * ct:128 * (ct + 1)]

    @pl.when(b == NBLK - 1)
    def _():
        red = _lane_tree_sum(acc_ref[...])
        m_ref[...] = red[:, 0:1] * _c(1e-05)


def _k_var(h_ref, m_ref, d_ref, acc_ref):
    b = pl.program_id(0)

    @pl.when(b == 0)
    def _():
        acc_ref[...] = jnp.zeros_like(acc_ref)

    d = h_ref[...] - m_ref[...]
    sq = d * d
    sq = jnp.where(_colmask(b, sq.shape), sq, _c(0.0))
    for ct in range(BLKC // 128):
        acc_ref[...] += sq[:, 128 * ct:128 * (ct + 1)]

    @pl.when(b == NBLK - 1)
    def _():
        red = _lane_tree_sum(acc_ref[...])
        var = red[:, 0:1] * _c(1e-05)
        d_ref[...] = jnp.sqrt(var + _c(1e-05))


def _k_mm2(h1_ref, m_ref, den_ref, g_ref, be_ref, w_ref, b2_ref,
           h2_ref, m2_ref, acc_ref):
    b = pl.program_id(0)
    x = (h1_ref[...] - m_ref[...]) / den_ref[...] * g_ref[...] + be_ref[...]
    gx = _gelu_xla(x)
    h2 = lax.dot_general(
        w_ref[...], gx, (((1,), (0,)), ((), ())),
        preferred_element_type=jnp.float32,
    )
    h2 = h2 + b2_ref[...]
    h2_ref[...] = h2

    @pl.when(b == 0)
    def _():
        acc_ref[...] = jnp.zeros_like(acc_ref)

    hm = jnp.where(_colmask(b, h2.shape), h2, _c(0.0))
    for ct in range(BLKC // 128):
        acc_ref[...] += hm[:, 128 * ct:128 * (ct + 1)]

    @pl.when(b == NBLK - 1)
    def _():
        red = _lane_tree_sum(acc_ref[...])
        m2_ref[...] = red[:, 0:1] * _c(1e-05)


def _k_scores(h2_ref, m_ref, den_ref, g_ref, be_ref, w3_ref, b3_ref,
              s_ref, mx_ref, acc_ref):
    b = pl.program_id(0)
    x = (h2_ref[...] - m_ref[...]) / den_ref[...] * g_ref[...] + be_ref[...]
    gx = _gelu_xla(x)
    u = lax.dot_general(
        w3_ref[...], gx, (((1,), (0,)), ((), ())),
        preferred_element_type=jnp.float32,
    )
    u = u + b3_ref[...]
    s = _c(1.0) / (jnp.exp(-u) + _c(1.0))
    s_ref[...] = s

    @pl.when(b == 0)
    def _():
        acc_ref[...] = jnp.full_like(acc_ref, -jnp.inf)

    sm = jnp.where(_colmask(b, s.shape), s, -jnp.inf)
    for ct in range(BLKC // 128):
        acc_ref[...] = jnp.maximum(acc_ref[...], sm[:, 128 * ct:128 * (ct + 1)])

    @pl.when(b == NBLK - 1)
    def _():
        red = _lane_tree_max(acc_ref[...])
        mx_ref[...] = red[:, 0:1]


def _k_sort(scores_ref, batch_ref, mx_ref, jt_ref, kt_ref, su_ref, sd_ref,
            ax_ref, ord_ref, keys_ref, idx_ref, pk_ref, pi_ref, iv_ref):
    rows = lax.broadcasted_iota(jnp.int32, (SROWS, 128), 0)
    cols = lax.broadcasted_iota(jnp.int32, (SROWS, 128), 1)
    ivec = rows * 128 + cols
    iv_ref[...] = ivec

    off = mx_ref[0] + _c(10.0)
    nreal_rows = NPAD // 128
    sc = scores_ref[...]
    bt = batch_ref[...].astype(jnp.float32)
    kreal = sc + bt * off
    kreal = jnp.where(ivec[:nreal_rows] < NPTS, kreal, jnp.inf)
    keys_ref[...] = jnp.full((SROWS, 128), jnp.inf, dtype=jnp.float32)
    keys_ref[0:nreal_rows, :] = kreal
    idx_ref[...] = ivec

    def stage(s, _):
        j = jt_ref[s]
        k = kt_ref[s]
        sup = su_ref[s]
        sdn = sd_ref[s]
        is_row = ax_ref[s]
        iv = iv_ref[...]
        maskj = (iv & j) == 0
        asc = (iv & k) == 0

        @pl.when(is_row == 1)
        def _():
            ks = keys_ref[...]
            ix = idx_ref[...]
            pk_ref[...] = jnp.where(
                maskj, pltpu.roll(ks, sup, 0), pltpu.roll(ks, sdn, 0))
            pi_ref[...] = jnp.where(
                maskj, pltpu.roll(ix, sup, 0), pltpu.roll(ix, sdn, 0))

        @pl.when(is_row == 0)
        def _():
            ks = keys_ref[...]
            ix = idx_ref[...]
            pk_ref[...] = jnp.where(
                maskj, pltpu.roll(ks, sup, 1), pltpu.roll(ks, sdn, 1))
            pi_ref[...] = jnp.where(
                maskj, pltpu.roll(ix, sup, 1), pltpu.roll(ix, sdn, 1))

        ks = keys_ref[...]
        ix = idx_ref[...]
        pk = pk_ref[...]
        pi = pi_ref[...]
        partner_less = (pk < ks) | ((pk == ks) & (pi < ix))
        want_min = ~(maskj ^ asc)
        take = ~(partner_less ^ want_min)
        keys_ref[...] = jnp.where(take, pk, ks)
        idx_ref[...] = jnp.where(take, pi, ix)
        return 0

    nstages = 17 * 18 // 2
    lax.fori_loop(0, nstages, stage, 0)
    ord_ref[...] = idx_ref[...]


def _make_inverse_sc():
    info = plsc.get_sparse_core_info()
    ns = info.num_subcores
    chunk = NSORT // ns
    mesh = plsc.VectorSubcoreMesh(core_axis_name="c", subcore_axis_name="s")

    @functools.partial(
        pl.kernel,
        mesh=mesh,
        out_type=jax.ShapeDtypeStruct((NSORT,), jnp.int32),
        scratch_types=[
            pltpu.VMEM((chunk,), jnp.int32),
            pltpu.VMEM((chunk,), jnp.int32),
            pltpu.VMEM_SHARED((NSORT,), jnp.int32),
            pltpu.SemaphoreType.DMA,
        ],
    )
    def inv_kernel(order_hbm, inv_hbm, idx_v, val_v, spmem, sem):
        cid = lax.axis_index("c")
        sid = lax.axis_index("s")

        @pl.when(cid == 0)
        def _():
            base = sid * chunk
            pltpu.sync_copy(order_hbm.at[pl.ds(base, chunk)], idx_v)

            def fill(i, _):
                val_v[pl.ds(i * 16, 16)] = (
                    lax.broadcasted_iota(jnp.int32, (16,), 0) + (base + i * 16)
                )
                return 0

            lax.fori_loop(0, chunk // 16, fill, 0)
            pltpu.async_copy(val_v, spmem.at[idx_v], sem).wait()
            plsc.subcore_barrier()
            pltpu.sync_copy(spmem.at[pl.ds(base, chunk)],
                            inv_hbm.at[pl.ds(base, chunk)])

    return inv_kernel


def _sort_tables():
    js, ks, sups, sdns, axs = [], [], [], [], []
    for klog in range(1, 18):
        for jlog in reversed(range(klog)):
            j = 1 << jlog
            k = 1 << klog
            js.append(j)
            ks.append(k)
            if j >= 128:
                jr = j // 128
                sups.append(SROWS - jr)
                sdns.append(jr)
                axs.append(1)
            else:
                sups.append(128 - j)
                sdns.append(j)
                axs.append(0)
    return (np.array(js, np.int32), np.array(ks, np.int32),
            np.array(sups, np.int32), np.array(sdns, np.int32),
            np.array(axs, np.int32))


def kernel(coord, feat, batch, W1, b1, g1, beta1, W2, b2, g2, beta2, W3, b3):
    xb = jnp.concatenate([coord, feat], axis=1).astype(jnp.bfloat16)
    xt = jnp.pad(xb.T, ((0, 0), (0, NPAD - NPTS)))
    w1t = W1.T
    w2t = W2.T
    w3t = W3.T
    b1c = b1[:, None]
    b2c = b2[:, None]
    g1c = g1[:, None]
    g2c = g2[:, None]
    be1c = beta1[:, None]
    be2c = beta2[:, None]
    b3c = b3[:, None]

    grid = (NBLK,)
    cspec = pl.BlockSpec((64, BLKC), lambda b: (0, b))
    full = pl.BlockSpec((64, 1), lambda b: (0, 0))

    h1t, m1 = pl.pallas_call(
        _k_mm1,
        grid=grid,
        in_specs=[
            pl.BlockSpec((131, BLKC), lambda b: (0, b)),
            pl.BlockSpec((64, 131), lambda b: (0, 0)),
            full,
        ],
        out_specs=[cspec, full],
        out_shape=[
            jax.ShapeDtypeStruct((64, NPAD), jnp.float32),
            jax.ShapeDtypeStruct((64, 1), jnp.float32),
        ],
        scratch_shapes=[pltpu.VMEM((64, 128), jnp.float32)],
    )(xt, w1t, b1c)

    den1 = pl.pallas_call(
        _k_var,
        grid=grid,
        in_specs=[cspec, full],
        out_specs=full,
        out_shape=jax.ShapeDtypeStruct((64, 1), jnp.float32),
        scratch_shapes=[pltpu.VMEM((64, 128), jnp.float32)],
    )(h1t, m1)

    h2t, m2 = pl.pallas_call(
        _k_mm2,
        grid=grid,
        in_specs=[
            cspec, full, full, full, full,
            pl.BlockSpec((64, 64), lambda b: (0, 0)),
            full,
        ],
        out_specs=[cspec, full],
        out_shape=[
            jax.ShapeDtypeStruct((64, NPAD), jnp.float32),
            jax.ShapeDtypeStruct((64, 1), jnp.float32),
        ],
        scratch_shapes=[pltpu.VMEM((64, 128), jnp.float32)],
    )(h1t, m1, den1, g1c, be1c, w2t, b2c)

    den2 = pl.pallas_call(
        _k_var,
        grid=grid,
        in_specs=[cspec, full],
        out_specs=full,
        out_shape=jax.ShapeDtypeStruct((64, 1), jnp.float32),
        scratch_shapes=[pltpu.VMEM((64, 128), jnp.float32)],
    )(h2t, m2)

    rspec = pl.BlockSpec((1, BLKC), lambda b: (0, b))
    rfull = pl.BlockSpec((1, 1), lambda b: (0, 0))
    scorest, mx = pl.pallas_call(
        _k_scores,
        grid=grid,
        in_specs=[
            cspec, full, full, full, full,
            pl.BlockSpec((1, 64), lambda b: (0, 0)),
            rfull,
        ],
        out_specs=[rspec, rfull],
        out_shape=[
            jax.ShapeDtypeStruct((1, NPAD), jnp.float32),
            jax.ShapeDtypeStruct((1, 1), jnp.float32),
        ],
        scratch_shapes=[pltpu.VMEM((1, 128), jnp.float32)],
    )(h2t, m2, den2, g2c, be2c, w3t, b3c)

    jt, kt, su, sd, ax = _sort_tables()
    batch_p = jnp.pad(batch, (0, NPAD - NPTS))
    nsr = NPAD // 128
    order2d = pl.pallas_call(
        _k_sort,
        in_specs=[
            pl.BlockSpec(memory_space=pltpu.VMEM),
            pl.BlockSpec(memory_space=pltpu.VMEM),
            pl.BlockSpec(memory_space=pltpu.SMEM),
            pl.BlockSpec(memory_space=pltpu.SMEM),
            pl.BlockSpec(memory_space=pltpu.SMEM),
            pl.BlockSpec(memory_space=pltpu.SMEM),
            pl.BlockSpec(memory_space=pltpu.SMEM),
            pl.BlockSpec(memory_space=pltpu.SMEM),
        ],
        out_shape=jax.ShapeDtypeStruct((SROWS, 128), jnp.int32),
        scratch_shapes=[
            pltpu.VMEM((SROWS, 128), jnp.float32),
            pltpu.VMEM((SROWS, 128), jnp.int32),
            pltpu.VMEM((SROWS, 128), jnp.float32),
            pltpu.VMEM((SROWS, 128), jnp.int32),
            pltpu.VMEM((SROWS, 128), jnp.int32),
        ],
    )(scorest.reshape(nsr, 128), batch_p.reshape(nsr, 128),
      jnp.asarray(mx).reshape(1), jt, kt, su, sd, ax)

    order_full = order2d.reshape(NSORT)
    inv_kernel = _make_inverse_sc()
    inv_full = inv_kernel(order_full)

    scores = scorest.reshape(NPAD)[:NPTS].reshape(NPTS, 1)
    orders = order_full[:NPTS].reshape(1, NPTS)
    inverses = inv_full[:NPTS].reshape(1, NPTS)
    return (scores, orders, inverses)
